# baseline (device time: 38424 ns/iter reference)
import jax
import jax.numpy as jnp
from jax import lax
from jax.experimental import pallas as pl
from jax.experimental.pallas import tpu as pltpu

Z = 4
BM = 512


def kernel(x, dy, gamma):
    m, d = x.shape
    n_blocks = m // BM

    def body(x_ref, dy_ref, gamma_ref, out_ref,
             acc_ref, comm_ref, send_sems, recv_sems):
        step = pl.program_id(0)
        my_x = lax.axis_index("x")
        my_y = lax.axis_index("y")
        my_z = lax.axis_index("z")

        xb = x_ref[...]
        dyb = dy_ref[...]
        ones_d = jnp.ones((d, 1), jnp.float32)
        s1 = jnp.dot(xb, ones_d, preferred_element_type=jnp.float32)
        s2 = jnp.dot(xb * xb, ones_d, preferred_element_type=jnp.float32)
        mu = s1 / d
        var = s2 / d - mu * mu
        rstd = lax.rsqrt(var + 1e-5)
        w2 = jnp.concatenate([mu * rstd, jnp.ones_like(mu)], axis=1).T
        dg1 = jnp.dot(rstd.T, dyb * xb, preferred_element_type=jnp.float32)
        p2 = jnp.dot(w2, dyb, preferred_element_type=jnp.float32)
        part = jnp.concatenate([dg1 - p2[0:1], p2[1:2]], axis=0)

        @pl.when(step == 0)
        def _():
            acc_ref[...] = part

        @pl.when(step != 0)
        def _():
            acc_ref[...] += part

        @pl.when(step == n_blocks - 1)
        def _():
            rdmas = []
            for k in range(1, Z):
                rdma = pltpu.make_async_remote_copy(
                    src_ref=acc_ref,
                    dst_ref=comm_ref.at[k - 1],
                    send_sem=send_sems.at[k - 1],
                    recv_sem=recv_sems.at[k - 1],
                    device_id=(my_x, my_y, (my_z + k) % Z),
                    device_id_type=pl.DeviceIdType.MESH,
                )
                rdma.start()
                rdmas.append(rdma)
            for rdma in rdmas:
                rdma.wait()
            out_ref[...] = (acc_ref[...] + comm_ref[0] + comm_ref[1]
                            + comm_ref[2])

    return pl.pallas_call(
        body,
        grid=(n_blocks,),
        out_shape=jax.ShapeDtypeStruct((2, d), jnp.float32),
        in_specs=[
            pl.BlockSpec((BM, d), lambda i: (i, 0)),
            pl.BlockSpec((BM, d), lambda i: (i, 0)),
            pl.BlockSpec(memory_space=pl.ANY),
        ],
        out_specs=pl.BlockSpec((2, d), lambda i: (0, 0)),
        scratch_shapes=[
            pltpu.VMEM((2, d), jnp.float32),
            pltpu.VMEM((Z - 1, 2, d), jnp.float32),
            pltpu.SemaphoreType.DMA((Z - 1,)),
            pltpu.SemaphoreType.DMA((Z - 1,)),
        ],
        compiler_params=pltpu.CompilerParams(
            dimension_semantics=("arbitrary",),
        ),
    )(x, dy, gamma)


# device time: 29754 ns/iter; 1.2914x vs baseline; 1.2914x over previous
import jax
import jax.numpy as jnp
from jax import lax
from jax.experimental import pallas as pl
from jax.experimental.pallas import tpu as pltpu

Z = 4
BM = 512


def kernel(x, dy, gamma):
    m, d = x.shape
    n_blocks = m // BM

    def body(x_ref, dy_ref, gamma_ref, out_ref,
             acc_ref, comm_ref, send_sems, recv_sems):
        step = pl.program_id(0)
        my_x = lax.axis_index("x")
        my_y = lax.axis_index("y")
        my_z = lax.axis_index("z")

        @pl.when(step == 0)
        def _():
            barrier_sem = pltpu.get_barrier_semaphore()
            for k in range(1, Z):
                pl.semaphore_signal(
                    barrier_sem, inc=1,
                    device_id=(my_x, my_y, (my_z + k) % Z),
                    device_id_type=pl.DeviceIdType.MESH,
                )

        xb = x_ref[...]
        dyb = dy_ref[...]
        mu = jnp.mean(xb, axis=1, keepdims=True)
        var = jnp.mean(xb * xb, axis=1, keepdims=True) - mu * mu
        xhat = (xb - mu) * lax.rsqrt(var + 1e-5)
        part = jnp.stack([jnp.sum(dyb * xhat, axis=0), jnp.sum(dyb, axis=0)])

        @pl.when(step == 0)
        def _():
            acc_ref[...] = part

        @pl.when(step != 0)
        def _():
            acc_ref[...] += part

        @pl.when(step == n_blocks - 1)
        def _():
            pl.semaphore_wait(pltpu.get_barrier_semaphore(), Z - 1)
            rdmas = []
            for k in range(1, Z):
                rdma = pltpu.make_async_remote_copy(
                    src_ref=acc_ref,
                    dst_ref=comm_ref.at[k - 1],
                    send_sem=send_sems.at[k - 1],
                    recv_sem=recv_sems.at[k - 1],
                    device_id=(my_x, my_y, (my_z + k) % Z),
                    device_id_type=pl.DeviceIdType.MESH,
                )
                rdma.start()
                rdmas.append(rdma)
            for rdma in rdmas:
                rdma.wait()
            out_ref[...] = (acc_ref[...] + comm_ref[0] + comm_ref[1]
                            + comm_ref[2])

    return pl.pallas_call(
        body,
        grid=(n_blocks,),
        out_shape=jax.ShapeDtypeStruct((2, d), jnp.float32),
        in_specs=[
            pl.BlockSpec((BM, d), lambda i: (i, 0)),
            pl.BlockSpec((BM, d), lambda i: (i, 0)),
            pl.BlockSpec(memory_space=pl.ANY),
        ],
        out_specs=pl.BlockSpec((2, d), lambda i: (0, 0)),
        scratch_shapes=[
            pltpu.VMEM((2, d), jnp.float32),
            pltpu.VMEM((Z - 1, 2, d), jnp.float32),
            pltpu.SemaphoreType.DMA((Z - 1,)),
            pltpu.SemaphoreType.DMA((Z - 1,)),
        ],
        compiler_params=pltpu.CompilerParams(
            dimension_semantics=("arbitrary",),
            collective_id=0,
        ),
    )(x, dy, gamma)
